# Initial kernel scaffold; baseline (speedup 1.0000x reference)
import jax
import jax.numpy as jnp
from jax.experimental import pallas as pl


def kernel(x, pos, batch, t, edge_index, params):
    raise NotImplementedError("write your pallas kernel here")


# speedup vs baseline: 2.3314x; 2.3314x over previous
import functools

import jax
import jax.numpy as jnp
import numpy as np
from jax import lax
from jax.experimental import pallas as pl
from jax.experimental.pallas import tpu as pltpu
from jax.experimental.pallas import tpu_sc as plsc

N = 10000
E = 320000
HID = 128
TDIM = 32
NG = 16
NRES = 20

NWORK = 32
EPW = E // NWORK
CHUNK = 80
NCH = EPW // CHUNK
ROWS_PER_SUB = 1000
ZROWS = 125

_f32 = jnp.float32


def _silu(v):
    return v * jax.nn.sigmoid(v)


def _sc_mesh():
    return plsc.VectorSubcoreMesh(core_axis_name="c", subcore_axis_name="s")


_DBG_JNP_GATHER = False
_DBG_JNP_SCATTER = False


def _gather_pair(table_a, table_b, src, dst, width):
    if _DBG_JNP_GATHER:
        return table_a[src] + table_b[dst]

    @functools.partial(
        pl.kernel,
        out_type=jax.ShapeDtypeStruct((E, width), _f32),
        mesh=_sc_mesh(),
        compiler_params=pltpu.CompilerParams(use_tc_tiling_on_sc=(width == HID)),
        scratch_types=[
            pltpu.VMEM((CHUNK,), jnp.int32),
            pltpu.VMEM((CHUNK,), jnp.int32),
            pltpu.VMEM((CHUNK, width), _f32),
            pltpu.VMEM((CHUNK, width), _f32),
            pltpu.SemaphoreType.DMA,
            pltpu.SemaphoreType.DMA,
        ],
    )
    def k(a_hbm, b_hbm, s_hbm, d_hbm, out_hbm, sidx, didx, abuf, bbuf, sem1, sem2):
        c = lax.axis_index("c")
        s = lax.axis_index("s")
        base = (s * 2 + c) * EPW

        def body(i, carry):
            off = base + i * CHUNK
            pltpu.sync_copy(s_hbm.at[pl.ds(off, CHUNK)], sidx)
            pltpu.sync_copy(d_hbm.at[pl.ds(off, CHUNK)], didx)
            cp1 = pltpu.make_async_copy(a_hbm.at[sidx], abuf, sem1)
            cp1.start()
            cp2 = pltpu.make_async_copy(b_hbm.at[didx], bbuf, sem2)
            cp2.start()
            cp1.wait()
            cp2.wait()

            def addrow(r, carry2):
                for j in range(width // 16):
                    sl = pl.ds(j * 16, 16)
                    abuf[r, sl] = abuf[r, sl] + bbuf[r, sl]
                return carry2

            lax.fori_loop(0, CHUNK, addrow, 0)
            pltpu.sync_copy(abuf, out_hbm.at[pl.ds(off, CHUNK), :])
            return carry

        lax.fori_loop(0, NCH, body, 0)

    return k(table_a, table_b, src, dst)


def _segment_scatter(rows, dst, width):
    if _DBG_JNP_SCATTER:
        full = jax.ops.segment_sum(rows, dst, num_segments=N)
        return jnp.stack([full, jnp.zeros_like(full)])

    @functools.partial(
        pl.kernel,
        out_type=jax.ShapeDtypeStruct((2, N, width), _f32),
        mesh=_sc_mesh(),
        compiler_params=pltpu.CompilerParams(use_tc_tiling_on_sc=(width == HID)),
        scratch_types=[
            pltpu.VMEM((CHUNK,), jnp.int32),
            pltpu.VMEM((CHUNK, width), _f32),
            pltpu.VMEM((ZROWS, width), _f32),
            pltpu.VMEM_SHARED((N, width), _f32),
        ],
    )
    def k(m_hbm, d_hbm, out_hbm, didx, mbuf, zbuf, agg_sh):
        c = lax.axis_index("c")
        s = lax.axis_index("s")

        def zrow(r, carry):
            for j in range(width // 16):
                zbuf[r, pl.ds(j * 16, 16)] = jnp.zeros((16,), _f32)
            return carry

        lax.fori_loop(0, ZROWS, zrow, 0)

        @pl.when(s < 10)
        def _zero():
            for kk in range(ROWS_PER_SUB // ZROWS):
                pltpu.sync_copy(
                    zbuf, agg_sh.at[pl.ds(s * ROWS_PER_SUB + kk * ZROWS, ZROWS), :]
                )

        plsc.subcore_barrier()

        base = (s * 2 + c) * EPW

        def body(i, carry):
            off = base + i * CHUNK
            pltpu.sync_copy(d_hbm.at[pl.ds(off, CHUNK)], didx)
            pltpu.sync_copy(m_hbm.at[pl.ds(off, CHUNK), :], mbuf)
            pltpu.sync_copy(mbuf, agg_sh.at[didx], add=True)
            return carry

        lax.fori_loop(0, NCH, body, 0)
        plsc.subcore_barrier()

        @pl.when(s < 10)
        def _copy_out():
            r0 = s * ROWS_PER_SUB
            pltpu.sync_copy(
                agg_sh.at[pl.ds(r0, ROWS_PER_SUB), :],
                out_hbm.at[c, pl.ds(r0, ROWS_PER_SUB), :],
            )

    return k(rows, dst)


_NB = 1000
_EB = 512


def _w_spec(shape):
    return pl.BlockSpec(shape, lambda i: (0,) * len(shape))


def _init_nodes(x2, b2, emb, te, iW1h, iW1t, ib1, iW2, ib2):
    def body(x_ref, b_ref, emb_ref, te_ref, w1h_ref, w1t_ref, b1_ref, w2_ref, b2_ref, o_ref):
        xv = x_ref[...]
        bv = b_ref[...]
        h0 = jnp.zeros((_NB, HID), _f32)
        for r in range(NRES):
            h0 = jnp.where(xv == r, emb_ref[pl.ds(r, 1), :], h0)
        ht = jnp.zeros((_NB, HID), _f32)
        for r in range(NG):
            ht = jnp.where(bv == r, te_ref[pl.ds(r, 1), :], ht)
        p = (
            jnp.dot(h0, w1h_ref[...], preferred_element_type=_f32)
            + jnp.dot(ht, w1t_ref[...], preferred_element_type=_f32)
            + b1_ref[...]
        )
        o_ref[...] = jnp.dot(_silu(p), w2_ref[...], preferred_element_type=_f32) + b2_ref[...]

    return pl.pallas_call(
        body,
        grid=(N // _NB,),
        in_specs=[
            pl.BlockSpec((_NB, 1), lambda i: (i, 0)),
            pl.BlockSpec((_NB, 1), lambda i: (i, 0)),
            _w_spec((NRES, HID)),
            _w_spec((NG, HID)),
            _w_spec((HID, HID)),
            _w_spec((HID, HID)),
            _w_spec((1, HID)),
            _w_spec((HID, HID)),
            _w_spec((1, HID)),
        ],
        out_specs=pl.BlockSpec((_NB, HID), lambda i: (i, 0)),
        out_shape=jax.ShapeDtypeStruct((N, HID), _f32),
    )(x2, b2, emb, te, iW1h, iW1t, ib1, iW2, ib2)


def _ab_tables(h, Ws, Wd, eb1):
    def body(h_ref, ws_ref, wd_ref, b1_ref, a_ref, b_ref):
        hv = h_ref[...]
        a_ref[...] = jnp.dot(hv, ws_ref[...], preferred_element_type=_f32) + b1_ref[...]
        b_ref[...] = jnp.dot(hv, wd_ref[...], preferred_element_type=_f32)

    return pl.pallas_call(
        body,
        grid=(N // _NB,),
        in_specs=[
            pl.BlockSpec((_NB, HID), lambda i: (i, 0)),
            _w_spec((HID, HID)),
            _w_spec((HID, HID)),
            _w_spec((1, HID)),
        ],
        out_specs=[
            pl.BlockSpec((_NB, HID), lambda i: (i, 0)),
            pl.BlockSpec((_NB, HID), lambda i: (i, 0)),
        ],
        out_shape=[
            jax.ShapeDtypeStruct((N, HID), _f32),
            jax.ShapeDtypeStruct((N, HID), _f32),
        ],
    )(h, Ws, Wd, eb1)


def _edge_mlp(pre, rel, ssum, wd, eW2, eb2, cWT, cb, last):
    def body(pre_ref, rel_ref, ss_ref, wd_ref, w2_ref, b2_ref, cwt_ref, cb_ref, *outs):
        relb = rel_ref[...]
        d2 = jnp.sum(relb * relb, axis=1, keepdims=True)
        d2b = d2.astype(jnp.bfloat16).astype(_f32)
        p = pre_ref[...] + d2b * wd_ref[...].astype(_f32)
        m1 = _silu(p)
        q = jnp.dot(m1, w2_ref[...], preferred_element_type=_f32) + b2_ref[...]
        m = _silu(q)
        s = jnp.sum(m * cwt_ref[...], axis=1, keepdims=True) + cb_ref[...]
        snew = ss_ref[...] + s
        if last:
            outs[0][...] = relb * snew
        else:
            outs[0][...] = m
            outs[1][...] = snew

    if last:
        out_specs = pl.BlockSpec((_EB, 16), lambda i: (i, 0))
        out_shape = jax.ShapeDtypeStruct((E, 16), _f32)
    else:
        out_specs = [
            pl.BlockSpec((_EB, HID), lambda i: (i, 0)),
            pl.BlockSpec((_EB, 1), lambda i: (i, 0)),
        ]
        out_shape = [
            jax.ShapeDtypeStruct((E, HID), _f32),
            jax.ShapeDtypeStruct((E, 1), _f32),
        ]
    return pl.pallas_call(
        body,
        grid=(E // _EB,),
        in_specs=[
            pl.BlockSpec((_EB, HID), lambda i: (i, 0)),
            pl.BlockSpec((_EB, 16), lambda i: (i, 0)),
            pl.BlockSpec((_EB, 1), lambda i: (i, 0)),
            _w_spec((1, HID)),
            _w_spec((HID, HID)),
            _w_spec((1, HID)),
            _w_spec((1, HID)),
            _w_spec((1, 1)),
        ],
        out_specs=out_specs,
        out_shape=out_shape,
    )(pre, rel, ssum, wd, eW2, eb2, cWT, cb)


def _node_update(h, agg2, nW1h, nW1a, nb1, nW2, nb2):
    def body(h_ref, agg_ref, w1h_ref, w1a_ref, b1_ref, w2_ref, b2_ref, o_ref):
        agg = agg_ref[0] + agg_ref[1]
        hv = h_ref[...]
        p = (
            jnp.dot(hv, w1h_ref[...], preferred_element_type=_f32)
            + jnp.dot(agg, w1a_ref[...], preferred_element_type=_f32)
            + b1_ref[...]
        )
        o_ref[...] = hv + jnp.dot(_silu(p), w2_ref[...], preferred_element_type=_f32) + b2_ref[...]

    return pl.pallas_call(
        body,
        grid=(N // _NB,),
        in_specs=[
            pl.BlockSpec((_NB, HID), lambda i: (i, 0)),
            pl.BlockSpec((2, _NB, HID), lambda i: (0, i, 0)),
            _w_spec((HID, HID)),
            _w_spec((HID, HID)),
            _w_spec((1, HID)),
            _w_spec((HID, HID)),
            _w_spec((1, HID)),
        ],
        out_specs=pl.BlockSpec((_NB, HID), lambda i: (i, 0)),
        out_shape=jax.ShapeDtypeStruct((N, HID), _f32),
    )(h, agg2, nW1h, nW1a, nb1, nW2, nb2)


def _final_ln(tp2, ln_w, ln_b):
    def body(tp_ref, w_ref, b_ref, o_ref):
        t3 = (tp_ref[0] + tp_ref[1])[:, :3]
        mu = jnp.mean(t3, axis=1, keepdims=True)
        var = jnp.mean((t3 - mu) * (t3 - mu), axis=1, keepdims=True)
        o_ref[...] = (t3 - mu) * lax.rsqrt(var + 1e-5) * w_ref[...] + b_ref[...]

    return pl.pallas_call(
        body,
        grid=(N // _NB,),
        in_specs=[
            pl.BlockSpec((2, _NB, 16), lambda i: (0, i, 0)),
            _w_spec((1, 3)),
            _w_spec((1, 3)),
        ],
        out_specs=pl.BlockSpec((_NB, 3), lambda i: (i, 0)),
        out_shape=jax.ShapeDtypeStruct((N, 3), _f32),
    )(tp2, ln_w, ln_b)


def _time_embed_small(t, p):
    half = TDIM // 2
    freq = jnp.exp(jnp.arange(half, dtype=_f32) * (-np.log(10000.0) / (half - 1)))
    e = t[:, None] * freq[None, :]
    te = jnp.concatenate([jnp.sin(e), jnp.cos(e)], axis=-1)
    return _silu(te @ p["tW1"] + p["tb1"]) @ p["tW2"] + p["tb2"]


_DBG_PURE_JNP = 0


def _bf16r(a):
    return a.astype(jnp.bfloat16).astype(_f32)


def _dbg_jnp_kernel(x, pos, batch, t, edge_index, params):
    import functools as _ft
    prec = "highest" if _DBG_PURE_JNP == 2 else None
    dot = _ft.partial(jnp.dot, precision=prec)
    mimic = _DBG_PURE_JNP == 3
    p = params
    src, dst = edge_index[0], edge_index[1]
    h = p['emb'][x]
    te = _time_embed_small(t, p)
    h = jnp.concatenate([h, te[batch]], axis=-1)
    h = _silu(dot(h, p['iW1']) + p['ib1'])
    h = dot(h, p['iW2']) + p['ib2']
    rel = pos[dst] - pos[src]
    d2 = jnp.sum(rel * rel, axis=-1, keepdims=True)
    ssum = jnp.zeros((E, 1), _f32)
    for lp in p['layers']:
        A = dot(h, lp['eW1'][:HID]) + lp['eb1']
        B = dot(h, lp['eW1'][HID:2 * HID])
        wd = lp['eW1'][2 * HID][None, :]
        if _DBG_PURE_JNP == 4:
            ef = jnp.concatenate([h[src], h[dst], d2], axis=-1)
            pre = dot(ef, lp['eW1']) + lp['eb1']
        elif _DBG_PURE_JNP == 5:
            hi_d = _bf16r(d2)
            lo_d = d2 - hi_d
            hi_w = _bf16r(wd)
            lo_w = wd - hi_w
            pre = A[src] + B[dst] + (hi_d * hi_w + hi_d * _bf16r(lo_w) + _bf16r(lo_d) * hi_w)
        elif mimic:
            pre = A[src] + B[dst] + _bf16r(d2) * _bf16r(wd)
        else:
            pre = A[src] + B[dst] + d2 * wd
        m = _silu(dot(_silu(pre), lp['eW2']) + lp['eb2'])
        agg = jax.ops.segment_sum(m, dst, num_segments=N)
        nf = jnp.concatenate([h, agg], axis=-1)
        h = h + _silu(dot(nf, lp['nW1']) + lp['nb1']) @ lp['nW2'] + lp['nb2']
        if mimic:
            s = jnp.sum(_bf16r(m) * _bf16r(lp['cW'].reshape(1, HID)), axis=1, keepdims=True)
            ssum = ssum + s + lp['cb']
        else:
            ssum = ssum + dot(m, lp['cW']) + lp['cb']
    total = jax.ops.segment_sum(rel * ssum, dst, num_segments=N)
    mu = jnp.mean(total, axis=-1, keepdims=True)
    var = jnp.var(total, axis=-1, keepdims=True)
    return (total - mu) / jnp.sqrt(var + 1e-5) * p['ln_w'] + p['ln_b']


def kernel(x, pos, batch, t, edge_index, params):
    if _DBG_PURE_JNP:
        return _dbg_jnp_kernel(x, pos, batch, t, edge_index, params)
    p = params
    src = edge_index[0].astype(jnp.int32)
    dst = edge_index[1].astype(jnp.int32)
    x2 = x.astype(jnp.int32).reshape(N, 1)
    b2 = batch.astype(jnp.int32).reshape(N, 1)

    te = _time_embed_small(t, p)

    h = _init_nodes(
        x2, b2, p["emb"], te,
        p["iW1"][:HID], p["iW1"][HID:], p["ib1"].reshape(1, HID),
        p["iW2"], p["ib2"].reshape(1, HID),
    )

    posp = jnp.pad(pos, ((0, 0), (0, 13)))
    rel = _gather_pair(-posp, posp, src, dst, 16)

    ssum = jnp.zeros((E, 1), _f32)
    rsp = None
    for li, lp in enumerate(p["layers"]):
        last = li == len(p["layers"]) - 1
        A, B = _ab_tables(h, lp["eW1"][:HID], lp["eW1"][HID : 2 * HID], lp["eb1"].reshape(1, HID))
        pre = _gather_pair(A, B, src, dst, HID)
        ew = (
            lp["eW1"][2 * HID].reshape(1, HID).astype(jnp.bfloat16),
            lp["eW2"],
            lp["eb2"].reshape(1, HID),
            lp["cW"].reshape(1, HID),
            lp["cb"].reshape(1, 1),
        )
        if last:
            rsp = _edge_mlp(pre, rel, ssum, *ew, True)
        else:
            m, ssum = _edge_mlp(pre, rel, ssum, *ew, False)
            agg2 = _segment_scatter(m, dst, HID)
            h = _node_update(
                h, agg2,
                lp["nW1"][:HID], lp["nW1"][HID:], lp["nb1"].reshape(1, HID),
                lp["nW2"], lp["nb2"].reshape(1, HID),
            )

    tp2 = _segment_scatter(rsp, dst, 16)
    return _final_ln(tp2, p["ln_w"].reshape(1, 3), p["ln_b"].reshape(1, 3))


# speedup vs baseline: 3.1250x; 1.3404x over previous
import functools

import jax
import jax.numpy as jnp
import numpy as np
from jax import lax
from jax.experimental import pallas as pl
from jax.experimental.pallas import tpu as pltpu
from jax.experimental.pallas import tpu_sc as plsc

N = 10000
E = 320000
HID = 128
TDIM = 32
NG = 16
NRES = 20

NWORK = 32
EPW = E // NWORK
CHUNK = 128
NCH = EPW // CHUNK
TAIL = EPW - NCH * CHUNK
ROWS_PER_SUB = 1000
ZROWS = 125

_f32 = jnp.float32


def _silu(v):
    return v * jax.nn.sigmoid(v)


def _sc_mesh():
    return plsc.VectorSubcoreMesh(core_axis_name="c", subcore_axis_name="s")


_DBG_JNP_GATHER = False
_DBG_JNP_SCATTER = False


def _gather_pair(table_a, table_b, src, dst, width):
    if _DBG_JNP_GATHER:
        return table_a[src] + table_b[dst]

    @functools.partial(
        pl.kernel,
        out_type=jax.ShapeDtypeStruct((E, width), _f32),
        mesh=_sc_mesh(),
        compiler_params=pltpu.CompilerParams(use_tc_tiling_on_sc=(width == HID)),
        scratch_types=[
            [pltpu.VMEM((CHUNK,), jnp.int32)] * 2,
            [pltpu.VMEM((CHUNK,), jnp.int32)] * 2,
            [pltpu.VMEM((CHUNK, width), _f32)] * 2,
            [pltpu.VMEM((CHUNK, width), _f32)] * 2,
            [pltpu.VMEM((CHUNK, width), _f32)] * 2,
            pltpu.VMEM((TAIL,), jnp.int32),
            pltpu.VMEM((TAIL,), jnp.int32),
            pltpu.VMEM((TAIL, width), _f32),
            pltpu.VMEM((TAIL, width), _f32),
            [pltpu.SemaphoreType.DMA] * 2,
            [pltpu.SemaphoreType.DMA] * 2,
            [pltpu.SemaphoreType.DMA] * 2,
            pltpu.SemaphoreType.DMA,
        ],
    )
    def k(a_hbm, b_hbm, s_hbm, d_hbm, out_hbm, sidx, didx, abuf, bbuf, obuf,
          sidx_t, didx_t, abuf_t, bbuf_t, gsem_a, gsem_b, wsem, tsem):
        c = lax.axis_index("c")
        s = lax.axis_index("s")
        base = (s * 2 + c) * EPW

        def start_gather(i, b):
            off = base + i * CHUNK
            pltpu.sync_copy(s_hbm.at[pl.ds(off, CHUNK)], sidx[b])
            pltpu.sync_copy(d_hbm.at[pl.ds(off, CHUNK)], didx[b])
            pltpu.make_async_copy(a_hbm.at[sidx[b]], abuf[b], gsem_a[b]).start()
            pltpu.make_async_copy(b_hbm.at[didx[b]], bbuf[b], gsem_b[b]).start()

        def finish_chunk(i, b, last):
            pltpu.make_async_copy(a_hbm.at[sidx[b]], abuf[b], gsem_a[b]).wait()
            pltpu.make_async_copy(b_hbm.at[didx[b]], bbuf[b], gsem_b[b]).wait()
            if not last:
                pltpu.make_async_copy(
                    obuf[b], out_hbm.at[pl.ds(base, CHUNK), :], wsem[b]
                ).wait()

            def addrow(r, carry2):
                for j in range(width // 16):
                    sl = pl.ds(j * 16, 16)
                    obuf[b][r, sl] = abuf[b][r, sl] + bbuf[b][r, sl]
                return carry2

            lax.fori_loop(0, CHUNK, addrow, 0)
            off = base + i * CHUNK
            pltpu.make_async_copy(
                obuf[b], out_hbm.at[pl.ds(off, CHUNK), :], wsem[b]
            ).start()

        for b in range(2):
            start_gather(b, b)
            pltpu.make_async_copy(
                obuf[b], out_hbm.at[pl.ds(base, CHUNK), :], wsem[b]
            ).start()

        def body(i, carry):
            for b in range(2):
                @pl.when(i % 2 == b)
                def _():
                    finish_chunk(i, b, False)
                    start_gather(i + 2, b)
            return carry

        lax.fori_loop(0, NCH - 2, body, 0)
        off_t = base + NCH * CHUNK
        pltpu.sync_copy(s_hbm.at[pl.ds(off_t, TAIL)], sidx_t)
        pltpu.sync_copy(d_hbm.at[pl.ds(off_t, TAIL)], didx_t)
        pltpu.make_async_copy(a_hbm.at[sidx_t], abuf_t, tsem).start()
        for b in range(2):
            finish_chunk(NCH - 2 + b, (NCH - 2 + b) % 2, False)
        pltpu.make_async_copy(a_hbm.at[sidx_t], abuf_t, tsem).wait()
        pltpu.make_async_copy(b_hbm.at[didx_t], bbuf_t, tsem).start()
        pltpu.make_async_copy(b_hbm.at[didx_t], bbuf_t, tsem).wait()

        def addrow_t(r, carry2):
            for j in range(width // 16):
                sl = pl.ds(j * 16, 16)
                abuf_t[r, sl] = abuf_t[r, sl] + bbuf_t[r, sl]
            return carry2

        lax.fori_loop(0, TAIL, addrow_t, 0)
        pltpu.sync_copy(abuf_t, out_hbm.at[pl.ds(off_t, TAIL), :])
        for b in range(2):
            pltpu.make_async_copy(
                obuf[b], out_hbm.at[pl.ds(base, CHUNK), :], wsem[b]
            ).wait()

    return k(table_a, table_b, src, dst)


def _segment_scatter(rows, dst, width):
    if _DBG_JNP_SCATTER:
        full = jax.ops.segment_sum(rows, dst, num_segments=N)
        return jnp.stack([full, jnp.zeros_like(full)])

    @functools.partial(
        pl.kernel,
        out_type=jax.ShapeDtypeStruct((2, N, width), _f32),
        mesh=_sc_mesh(),
        compiler_params=pltpu.CompilerParams(use_tc_tiling_on_sc=(width == HID)),
        scratch_types=[
            [pltpu.VMEM((CHUNK,), jnp.int32)] * 2,
            [pltpu.VMEM((CHUNK, width), _f32)] * 2,
            pltpu.VMEM((TAIL,), jnp.int32),
            pltpu.VMEM((TAIL, width), _f32),
            pltpu.VMEM_SHARED((N, width), _f32),
            [pltpu.SemaphoreType.DMA] * 2,
            [pltpu.SemaphoreType.DMA] * 2,
            pltpu.SemaphoreType.DMA,
        ],
    )
    def k(m_hbm, d_hbm, out_hbm, didx, mbuf, didx_t, mbuf_t, agg_sh,
          lsem, ssem, tsem):
        c = lax.axis_index("c")
        s = lax.axis_index("s")

        def zrow(r, carry):
            for j in range(width // 16):
                mbuf[0][r, pl.ds(j * 16, 16)] = jnp.zeros((16,), _f32)
            return carry

        lax.fori_loop(0, ZROWS, zrow, 0)

        @pl.when(s < 10)
        def _zero():
            for kk in range(ROWS_PER_SUB // ZROWS):
                pltpu.sync_copy(
                    mbuf[0].at[pl.ds(0, ZROWS), :],
                    agg_sh.at[pl.ds(s * ROWS_PER_SUB + kk * ZROWS, ZROWS), :],
                )

        plsc.subcore_barrier()

        base = (s * 2 + c) * EPW

        def start_load(i, b):
            off = base + i * CHUNK
            pltpu.sync_copy(d_hbm.at[pl.ds(off, CHUNK)], didx[b])
            pltpu.make_async_copy(
                m_hbm.at[pl.ds(off, CHUNK), :], mbuf[b], lsem[b]
            ).start()

        def wait_load(b):
            pltpu.make_async_copy(
                m_hbm.at[pl.ds(base, CHUNK), :], mbuf[b], lsem[b]
            ).wait()

        def start_scatter(b):
            pltpu.make_async_copy(mbuf[b], agg_sh.at[didx[b]], ssem[b]).start(add=True)

        def wait_scatter(b):
            pltpu.make_async_copy(mbuf[b], agg_sh.at[didx[b]], ssem[b]).wait()

        def body(i, carry):
            for b in range(2):
                @pl.when(i % 2 == b)
                def _():
                    @pl.when(i >= 2)
                    def _w():
                        wait_scatter(b)

                    start_load(i, b)
                    wait_load(b)
                    start_scatter(b)
            return carry

        lax.fori_loop(0, NCH, body, 0)
        off_t = base + NCH * CHUNK
        pltpu.sync_copy(d_hbm.at[pl.ds(off_t, TAIL)], didx_t)
        pltpu.sync_copy(m_hbm.at[pl.ds(off_t, TAIL), :], mbuf_t)
        pltpu.make_async_copy(mbuf_t, agg_sh.at[didx_t], tsem).start(add=True)
        for b in range(2):
            wait_scatter(b)
        pltpu.make_async_copy(mbuf_t, agg_sh.at[didx_t], tsem).wait()
        plsc.subcore_barrier()

        @pl.when(s < 10)
        def _copy_out():
            r0 = s * ROWS_PER_SUB
            pltpu.sync_copy(
                agg_sh.at[pl.ds(r0, ROWS_PER_SUB), :],
                out_hbm.at[c, pl.ds(r0, ROWS_PER_SUB), :],
            )

    return k(rows, dst)


_NB = 1000
_EB = 512


def _w_spec(shape):
    return pl.BlockSpec(shape, lambda i: (0,) * len(shape))


def _init_nodes(x2, b2, emb, te, iW1h, iW1t, ib1, iW2, ib2):
    def body(x_ref, b_ref, emb_ref, te_ref, w1h_ref, w1t_ref, b1_ref, w2_ref, b2_ref, o_ref):
        xv = x_ref[...]
        bv = b_ref[...]
        h0 = jnp.zeros((_NB, HID), _f32)
        for r in range(NRES):
            h0 = jnp.where(xv == r, emb_ref[pl.ds(r, 1), :], h0)
        ht = jnp.zeros((_NB, HID), _f32)
        for r in range(NG):
            ht = jnp.where(bv == r, te_ref[pl.ds(r, 1), :], ht)
        p = (
            jnp.dot(h0, w1h_ref[...], preferred_element_type=_f32)
            + jnp.dot(ht, w1t_ref[...], preferred_element_type=_f32)
            + b1_ref[...]
        )
        o_ref[...] = jnp.dot(_silu(p), w2_ref[...], preferred_element_type=_f32) + b2_ref[...]

    return pl.pallas_call(
        body,
        grid=(N // _NB,),
        in_specs=[
            pl.BlockSpec((_NB, 1), lambda i: (i, 0)),
            pl.BlockSpec((_NB, 1), lambda i: (i, 0)),
            _w_spec((NRES, HID)),
            _w_spec((NG, HID)),
            _w_spec((HID, HID)),
            _w_spec((HID, HID)),
            _w_spec((1, HID)),
            _w_spec((HID, HID)),
            _w_spec((1, HID)),
        ],
        out_specs=pl.BlockSpec((_NB, HID), lambda i: (i, 0)),
        out_shape=jax.ShapeDtypeStruct((N, HID), _f32),
    )(x2, b2, emb, te, iW1h, iW1t, ib1, iW2, ib2)


def _ab_tables(h, Ws, Wd, eb1):
    def body(h_ref, ws_ref, wd_ref, b1_ref, a_ref, b_ref):
        hv = h_ref[...]
        a_ref[...] = jnp.dot(hv, ws_ref[...], preferred_element_type=_f32) + b1_ref[...]
        b_ref[...] = jnp.dot(hv, wd_ref[...], preferred_element_type=_f32)

    return pl.pallas_call(
        body,
        grid=(N // _NB,),
        in_specs=[
            pl.BlockSpec((_NB, HID), lambda i: (i, 0)),
            _w_spec((HID, HID)),
            _w_spec((HID, HID)),
            _w_spec((1, HID)),
        ],
        out_specs=[
            pl.BlockSpec((_NB, HID), lambda i: (i, 0)),
            pl.BlockSpec((_NB, HID), lambda i: (i, 0)),
        ],
        out_shape=[
            jax.ShapeDtypeStruct((N, HID), _f32),
            jax.ShapeDtypeStruct((N, HID), _f32),
        ],
    )(h, Ws, Wd, eb1)


def _edge_mlp(pre, rel, ssum, wd, eW2, eb2, cWT, cb, last):
    def body(pre_ref, rel_ref, ss_ref, wd_ref, w2_ref, b2_ref, cwt_ref, cb_ref, *outs):
        relb = rel_ref[...]
        d2 = jnp.sum(relb * relb, axis=1, keepdims=True)
        d2b = d2.astype(jnp.bfloat16).astype(_f32)
        p = pre_ref[...] + d2b * wd_ref[...].astype(_f32)
        m1 = _silu(p)
        q = jnp.dot(m1, w2_ref[...], preferred_element_type=_f32) + b2_ref[...]
        m = _silu(q)
        s = jnp.sum(m * cwt_ref[...], axis=1, keepdims=True) + cb_ref[...]
        snew = ss_ref[...] + s
        if last:
            outs[0][...] = relb * snew
        else:
            outs[0][...] = m
            outs[1][...] = snew

    if last:
        out_specs = pl.BlockSpec((_EB, 16), lambda i: (i, 0))
        out_shape = jax.ShapeDtypeStruct((E, 16), _f32)
    else:
        out_specs = [
            pl.BlockSpec((_EB, HID), lambda i: (i, 0)),
            pl.BlockSpec((_EB, 1), lambda i: (i, 0)),
        ]
        out_shape = [
            jax.ShapeDtypeStruct((E, HID), _f32),
            jax.ShapeDtypeStruct((E, 1), _f32),
        ]
    return pl.pallas_call(
        body,
        grid=(E // _EB,),
        in_specs=[
            pl.BlockSpec((_EB, HID), lambda i: (i, 0)),
            pl.BlockSpec((_EB, 16), lambda i: (i, 0)),
            pl.BlockSpec((_EB, 1), lambda i: (i, 0)),
            _w_spec((1, HID)),
            _w_spec((HID, HID)),
            _w_spec((1, HID)),
            _w_spec((1, HID)),
            _w_spec((1, 1)),
        ],
        out_specs=out_specs,
        out_shape=out_shape,
    )(pre, rel, ssum, wd, eW2, eb2, cWT, cb)


def _node_update(h, agg2, nW1h, nW1a, nb1, nW2, nb2):
    def body(h_ref, agg_ref, w1h_ref, w1a_ref, b1_ref, w2_ref, b2_ref, o_ref):
        agg = agg_ref[0] + agg_ref[1]
        hv = h_ref[...]
        p = (
            jnp.dot(hv, w1h_ref[...], preferred_element_type=_f32)
            + jnp.dot(agg, w1a_ref[...], preferred_element_type=_f32)
            + b1_ref[...]
        )
        o_ref[...] = hv + jnp.dot(_silu(p), w2_ref[...], preferred_element_type=_f32) + b2_ref[...]

    return pl.pallas_call(
        body,
        grid=(N // _NB,),
        in_specs=[
            pl.BlockSpec((_NB, HID), lambda i: (i, 0)),
            pl.BlockSpec((2, _NB, HID), lambda i: (0, i, 0)),
            _w_spec((HID, HID)),
            _w_spec((HID, HID)),
            _w_spec((1, HID)),
            _w_spec((HID, HID)),
            _w_spec((1, HID)),
        ],
        out_specs=pl.BlockSpec((_NB, HID), lambda i: (i, 0)),
        out_shape=jax.ShapeDtypeStruct((N, HID), _f32),
    )(h, agg2, nW1h, nW1a, nb1, nW2, nb2)


def _final_ln(tp2, ln_w, ln_b):
    def body(tp_ref, w_ref, b_ref, o_ref):
        t3 = (tp_ref[0] + tp_ref[1])[:, :3]
        mu = jnp.mean(t3, axis=1, keepdims=True)
        var = jnp.mean((t3 - mu) * (t3 - mu), axis=1, keepdims=True)
        o_ref[...] = (t3 - mu) * lax.rsqrt(var + 1e-5) * w_ref[...] + b_ref[...]

    return pl.pallas_call(
        body,
        grid=(N // _NB,),
        in_specs=[
            pl.BlockSpec((2, _NB, 16), lambda i: (0, i, 0)),
            _w_spec((1, 3)),
            _w_spec((1, 3)),
        ],
        out_specs=pl.BlockSpec((_NB, 3), lambda i: (i, 0)),
        out_shape=jax.ShapeDtypeStruct((N, 3), _f32),
    )(tp2, ln_w, ln_b)


def _time_embed_small(t, p):
    half = TDIM // 2
    freq = jnp.exp(jnp.arange(half, dtype=_f32) * (-np.log(10000.0) / (half - 1)))
    e = t[:, None] * freq[None, :]
    te = jnp.concatenate([jnp.sin(e), jnp.cos(e)], axis=-1)
    return _silu(te @ p["tW1"] + p["tb1"]) @ p["tW2"] + p["tb2"]


_DBG_PURE_JNP = 0


def _bf16r(a):
    return a.astype(jnp.bfloat16).astype(_f32)


def _dbg_jnp_kernel(x, pos, batch, t, edge_index, params):
    import functools as _ft
    prec = "highest" if _DBG_PURE_JNP == 2 else None
    dot = _ft.partial(jnp.dot, precision=prec)
    mimic = _DBG_PURE_JNP == 3
    p = params
    src, dst = edge_index[0], edge_index[1]
    h = p['emb'][x]
    te = _time_embed_small(t, p)
    h = jnp.concatenate([h, te[batch]], axis=-1)
    h = _silu(dot(h, p['iW1']) + p['ib1'])
    h = dot(h, p['iW2']) + p['ib2']
    rel = pos[dst] - pos[src]
    d2 = jnp.sum(rel * rel, axis=-1, keepdims=True)
    ssum = jnp.zeros((E, 1), _f32)
    for lp in p['layers']:
        A = dot(h, lp['eW1'][:HID]) + lp['eb1']
        B = dot(h, lp['eW1'][HID:2 * HID])
        wd = lp['eW1'][2 * HID][None, :]
        if _DBG_PURE_JNP == 4:
            ef = jnp.concatenate([h[src], h[dst], d2], axis=-1)
            pre = dot(ef, lp['eW1']) + lp['eb1']
        elif _DBG_PURE_JNP == 5:
            hi_d = _bf16r(d2)
            lo_d = d2 - hi_d
            hi_w = _bf16r(wd)
            lo_w = wd - hi_w
            pre = A[src] + B[dst] + (hi_d * hi_w + hi_d * _bf16r(lo_w) + _bf16r(lo_d) * hi_w)
        elif mimic:
            pre = A[src] + B[dst] + _bf16r(d2) * _bf16r(wd)
        else:
            pre = A[src] + B[dst] + d2 * wd
        m = _silu(dot(_silu(pre), lp['eW2']) + lp['eb2'])
        agg = jax.ops.segment_sum(m, dst, num_segments=N)
        nf = jnp.concatenate([h, agg], axis=-1)
        h = h + _silu(dot(nf, lp['nW1']) + lp['nb1']) @ lp['nW2'] + lp['nb2']
        if mimic:
            s = jnp.sum(_bf16r(m) * _bf16r(lp['cW'].reshape(1, HID)), axis=1, keepdims=True)
            ssum = ssum + s + lp['cb']
        else:
            ssum = ssum + dot(m, lp['cW']) + lp['cb']
    total = jax.ops.segment_sum(rel * ssum, dst, num_segments=N)
    mu = jnp.mean(total, axis=-1, keepdims=True)
    var = jnp.var(total, axis=-1, keepdims=True)
    return (total - mu) / jnp.sqrt(var + 1e-5) * p['ln_w'] + p['ln_b']


def kernel(x, pos, batch, t, edge_index, params):
    if _DBG_PURE_JNP:
        return _dbg_jnp_kernel(x, pos, batch, t, edge_index, params)
    p = params
    src = edge_index[0].astype(jnp.int32)
    dst = edge_index[1].astype(jnp.int32)
    x2 = x.astype(jnp.int32).reshape(N, 1)
    b2 = batch.astype(jnp.int32).reshape(N, 1)

    te = _time_embed_small(t, p)

    h = _init_nodes(
        x2, b2, p["emb"], te,
        p["iW1"][:HID], p["iW1"][HID:], p["ib1"].reshape(1, HID),
        p["iW2"], p["ib2"].reshape(1, HID),
    )

    posp = jnp.pad(pos, ((0, 0), (0, 13)))
    rel = _gather_pair(-posp, posp, src, dst, 16)

    ssum = jnp.zeros((E, 1), _f32)
    rsp = None
    for li, lp in enumerate(p["layers"]):
        last = li == len(p["layers"]) - 1
        A, B = _ab_tables(h, lp["eW1"][:HID], lp["eW1"][HID : 2 * HID], lp["eb1"].reshape(1, HID))
        pre = _gather_pair(A, B, src, dst, HID)
        ew = (
            lp["eW1"][2 * HID].reshape(1, HID).astype(jnp.bfloat16),
            lp["eW2"],
            lp["eb2"].reshape(1, HID),
            lp["cW"].reshape(1, HID),
            lp["cb"].reshape(1, 1),
        )
        if last:
            rsp = _edge_mlp(pre, rel, ssum, *ew, True)
        else:
            m, ssum = _edge_mlp(pre, rel, ssum, *ew, False)
            agg2 = _segment_scatter(m, dst, HID)
            h = _node_update(
                h, agg2,
                lp["nW1"][:HID], lp["nW1"][HID:], lp["nb1"].reshape(1, HID),
                lp["nW2"], lp["nb2"].reshape(1, HID),
            )

    tp2 = _segment_scatter(rsp, dst, 16)
    return _final_ln(tp2, p["ln_w"].reshape(1, 3), p["ln_b"].reshape(1, 3))
